# baseline (speedup 1.0000x reference)
import functools

import jax
import jax.numpy as jnp
from jax import lax
from jax.experimental import pallas as pl
from jax.experimental.pallas import tpu as pltpu
from jax.experimental.pallas import tpu_sc as plsc

N = 10000
D = 256
C = 7
A4 = 0.1

NC = 2
NS = 16
NW = NC * NS
CH = 16
NCHUNK = N // CH
_FULL = NCHUNK - (NCHUNK // NW) * NW


def _sc_body(seq_hbm, seq1_hbm, lab_hbm, svec_hbm,
             raw_hbm, sums_hbm, cnt_hbm,
             seq_v, seq1_v, raw_v, lab_v, s_v, sums_v, cnt_v):
    wid = lax.axis_index("s") * NC + lax.axis_index("c")
    lanes = lax.iota(jnp.int32, 16)
    zero16 = jnp.zeros((16,), jnp.float32)

    for t in range(D * C // 16):
        sums_v[pl.ds(t * 16, 16)] = zero16
    cnt_v[...] = zero16

    pltpu.sync_copy(svec_hbm, s_v)

    nk = jnp.where(wid < _FULL, NCHUNK // NW + 1, NCHUNK // NW).astype(jnp.int32)

    def chunk_body(i, _):
        cidx = wid + NW * i
        base = cidx * CH
        off = base * D
        pltpu.sync_copy(seq_hbm.at[pl.ds(off, CH * D)], seq_v)
        pltpu.sync_copy(seq1_hbm.at[pl.ds(off, CH * D)], seq1_v)
        pltpu.sync_copy(lab_hbm.at[pl.ds(base, CH)], lab_v)
        cvec = jnp.zeros((16,), jnp.float32)
        lv16 = lab_v[...]
        for r in range(CH):
            lab_r = lv16[r]
            lvec = jnp.full((16,), lab_r, jnp.int32)
            cvec = cvec + jnp.where(lanes == lvec, 1.0, 0.0)
            rowbase = lvec * D
            for j in range(D // 16):
                x = seq_v[pl.ds(r * D + j * 16, 16)]
                x1 = seq1_v[pl.ds(r * D + j * 16, 16)]
                s = s_v[pl.ds(j * 16, 16)]
                y = x * s
                e = jnp.exp(jnp.minimum(y, 0.0)) - 1.0
                z = jnp.where(y > 0.0, y, e) + A4 * x1
                raw_v[pl.ds(r * D + j * 16, 16)] = z
                plsc.addupdate_scatter(sums_v.at[...],
                                       [rowbase + (lanes + j * 16)], z)
        cnt_v[...] = cnt_v[...] + cvec
        pltpu.sync_copy(raw_v, raw_hbm.at[pl.ds(off, CH * D)])
        return ()

    lax.fori_loop(0, nk, chunk_body, ())

    pltpu.sync_copy(sums_v, sums_hbm.at[wid])
    pltpu.sync_copy(cnt_v, cnt_hbm.at[wid])


_sc_phase = functools.partial(
    pl.kernel,
    out_type=[
        jax.ShapeDtypeStruct((N * D,), jnp.float32),
        jax.ShapeDtypeStruct((NW, C * D), jnp.float32),
        jax.ShapeDtypeStruct((NW, 16), jnp.float32),
    ],
    mesh=plsc.VectorSubcoreMesh(core_axis_name="c", subcore_axis_name="s"),
    compiler_params=pltpu.CompilerParams(needs_layout_passes=False),
    scratch_types=[
        pltpu.VMEM((CH * D,), jnp.float32),
        pltpu.VMEM((CH * D,), jnp.float32),
        pltpu.VMEM((CH * D,), jnp.float32),
        pltpu.VMEM((CH,), jnp.int32),
        pltpu.VMEM((D,), jnp.float32),
        pltpu.VMEM((C * D,), jnp.float32),
        pltpu.VMEM((16,), jnp.float32),
    ],
)(_sc_body)


def _tc_body(raw_ref, sums_ref, cnt_ref, out_ref):
    raw = raw_ref[...]
    s32 = sums_ref[...]
    c32 = cnt_ref[...]
    sums = s32[0:C, :]
    for w in range(1, NW):
        sums = sums + s32[w * C:(w + 1) * C, :]
    counts = jnp.sum(c32, axis=0)[0:C]
    ave = sums / jnp.maximum(counts, 1.0)[:, None]
    an = jnp.maximum(jnp.sqrt(jnp.sum(ave * ave, axis=1)), 1e-8)
    rn = jnp.maximum(jnp.sqrt(jnp.sum(raw * raw, axis=1, keepdims=True)), 1e-8)
    ret = lax.dot_general(raw, ave, (((1,), (1,)), ((), ())),
                          preferred_element_type=jnp.float32)
    ret = ret / (rn * an[None, :])
    m = jnp.max(ret, axis=1, keepdims=True)
    e = jnp.exp(ret - m)
    out_ref[...] = e / jnp.sum(e, axis=1, keepdims=True)


def kernel(seq, seq1, labels, prompt1, prompt2, prompt3,
           wp_weight, dff_weight, dp_weight):
    prompt = jnp.concatenate([prompt1, prompt2, prompt3], axis=0)
    w = 1.0 + jax.nn.elu(jnp.matmul(wp_weight, prompt))
    svec = (dff_weight[0, 0] * w + dff_weight[0, 1] * dp_weight)[0]

    raw_flat, sums32, cnt32 = _sc_phase(
        seq.reshape(-1), seq1.reshape(-1), labels, svec)

    out = pl.pallas_call(
        _tc_body,
        out_shape=jax.ShapeDtypeStruct((N, C), jnp.float32),
    )(raw_flat.reshape(N, D), sums32.reshape(NW * C, D), cnt32)
    return out


# speedup vs baseline: 2.9549x; 2.9549x over previous
import functools

import jax
import jax.numpy as jnp
from jax import lax
from jax.experimental import pallas as pl
from jax.experimental.pallas import tpu as pltpu
from jax.experimental.pallas import tpu_sc as plsc

N = 10000
D = 256
C = 7
A4 = 0.1

NC = 2
NS = 16
NW = NC * NS
CH = 80
NCHUNK = N // CH
MAXK = -(-NCHUNK // NW)
_FULL = NCHUNK - (NCHUNK // NW) * NW


def _ew_body(seq_ref, seq1_ref, s_ref, raw_ref):
    x = seq_ref[...]
    y = x * s_ref[...]
    e = jnp.exp(jnp.minimum(y, 0.0)) - 1.0
    raw_ref[...] = jnp.where(y > 0.0, y, e) + A4 * seq1_ref[...]


def _stage_a(seq, seq1, svec2d):
    blk = N // 10
    return pl.pallas_call(
        _ew_body,
        grid=(10,),
        in_specs=[
            pl.BlockSpec((blk, D), lambda i: (i, 0)),
            pl.BlockSpec((blk, D), lambda i: (i, 0)),
            pl.BlockSpec((1, D), lambda i: (0, 0)),
        ],
        out_specs=pl.BlockSpec((blk, D), lambda i: (i, 0)),
        out_shape=jax.ShapeDtypeStruct((N, D), jnp.float32),
    )(seq, seq1, svec2d)


def _sc_body(raw_hbm, lab_hbm,
             sums_out, cnt_out,
             raw0, raw1, lab0, lab1, sums_v, cnt_v,
             sr0, sr1, sl0, sl1):
    wid = lax.axis_index("s") * NC + lax.axis_index("c")
    raws = (raw0, raw1)
    labs = (lab0, lab1)
    srs = (sr0, sr1)
    sls = (sl0, sl1)
    lanes = lax.iota(jnp.int32, 16)
    zero16 = jnp.zeros((16,), jnp.float32)

    for t in range(C * D // 16):
        sums_v[pl.ds(t * 16, 16)] = zero16
    cnt_v[...] = zero16

    def issue(i):
        slot = i % 2
        base = (wid + NW * i) * CH
        pltpu.async_copy(raw_hbm.at[pl.ds(base * D, CH * D)], raws[slot],
                         srs[slot])
        pltpu.async_copy(lab_hbm.at[pl.ds(base, CH)], labs[slot], sls[slot])

    def wait(i):
        slot = i % 2
        base = (wid + NW * i) * CH
        pltpu.make_async_copy(raw_hbm.at[pl.ds(base * D, CH * D)],
                              raws[slot], srs[slot]).wait()
        pltpu.make_async_copy(lab_hbm.at[pl.ds(base, CH)], labs[slot],
                              sls[slot]).wait()

    def process(i):
        slot = i % 2
        raw_v = raws[slot]
        lab_v = labs[slot]

        def row(r, cvec):
            lvec = plsc.load_gather(lab_v.at[...], [jnp.full((16,), r)])
            cvec = cvec + jnp.where(lanes == lvec, 1.0, 0.0)
            badd = lvec * D + lanes
            roff = r * D
            for j in range(D // 16):
                z = raw_v[pl.ds(roff + j * 16, 16)]
                plsc.addupdate_scatter(sums_v.at[...], [badd + j * 16], z)
            return cvec

        cvec = lax.fori_loop(0, CH, row, jnp.zeros((16,), jnp.float32))
        cnt_v[...] = cnt_v[...] + cvec

    issue(0)
    for i in range(MAXK - 1):
        wait(i)
        if i + 1 < MAXK - 1:
            issue(i + 1)
        else:
            @pl.when(wid < _FULL)
            def _():
                issue(MAXK - 1)
        process(i)

    @pl.when(wid < _FULL)
    def _():
        wait(MAXK - 1)
        process(MAXK - 1)

    pltpu.sync_copy(sums_v, sums_out.at[wid])
    pltpu.sync_copy(cnt_v, cnt_out.at[wid])


_stage_b = functools.partial(
    pl.kernel,
    out_type=[
        jax.ShapeDtypeStruct((NW, C * D), jnp.float32),
        jax.ShapeDtypeStruct((NW, 16), jnp.float32),
    ],
    mesh=plsc.VectorSubcoreMesh(core_axis_name="c", subcore_axis_name="s"),
    compiler_params=pltpu.CompilerParams(needs_layout_passes=False),
    scratch_types=[
        pltpu.VMEM((CH * D,), jnp.float32),
        pltpu.VMEM((CH * D,), jnp.float32),
        pltpu.VMEM((CH,), jnp.int32),
        pltpu.VMEM((CH,), jnp.int32),
        pltpu.VMEM((C * D,), jnp.float32),
        pltpu.VMEM((16,), jnp.float32),
        pltpu.SemaphoreType.DMA,
        pltpu.SemaphoreType.DMA,
        pltpu.SemaphoreType.DMA,
        pltpu.SemaphoreType.DMA,
    ],
)(_sc_body)


def _head_body(raw_ref, sums_ref, cnt_ref, out_ref):
    raw = raw_ref[...]
    s32 = sums_ref[...]
    c32 = cnt_ref[...]
    sums = s32[0:C, :]
    for w in range(1, NW):
        sums = sums + s32[w * C:(w + 1) * C, :]
    counts = jnp.sum(c32, axis=0)[0:C]
    ave = sums / jnp.maximum(counts, 1.0)[:, None]
    an = jnp.maximum(jnp.sqrt(jnp.sum(ave * ave, axis=1)), 1e-8)
    rn = jnp.maximum(jnp.sqrt(jnp.sum(raw * raw, axis=1, keepdims=True)), 1e-8)
    ret = lax.dot_general(raw, ave, (((1,), (1,)), ((), ())),
                          preferred_element_type=jnp.float32)
    ret = ret / (rn * an[None, :])
    m = jnp.max(ret, axis=1, keepdims=True)
    e = jnp.exp(ret - m)
    out_ref[...] = e / jnp.sum(e, axis=1, keepdims=True)


def kernel(seq, seq1, labels, prompt1, prompt2, prompt3,
           wp_weight, dff_weight, dp_weight):
    prompt = jnp.concatenate([prompt1, prompt2, prompt3], axis=0)
    w = 1.0 + jax.nn.elu(jnp.matmul(wp_weight, prompt))
    svec2d = dff_weight[0, 0] * w + dff_weight[0, 1] * dp_weight

    raw = _stage_a(seq, seq1, svec2d)

    sums32, cnt32 = _stage_b(raw.reshape(-1), labels)

    out = pl.pallas_call(
        _head_body,
        out_shape=jax.ShapeDtypeStruct((N, C), jnp.float32),
    )(raw, sums32.reshape(NW * C, D), cnt32)
    return out


# speedup vs baseline: 3.6579x; 1.2379x over previous
import functools

import jax
import jax.numpy as jnp
from jax import lax
from jax.experimental import pallas as pl
from jax.experimental.pallas import tpu as pltpu
from jax.experimental.pallas import tpu_sc as plsc

N = 10000
D = 256
C = 7
A4 = 0.1

NC = 2
NS = 16
NW = NC * NS
CP = 8
CH = 80
NCHUNK = N // CH
MAXK = -(-NCHUNK // NW)
_FULL = NCHUNK - (NCHUNK // NW) * NW

LBLK = 16


def _ew_body(seq_ref, seq1_ref, s_ref, raw_ref):
    x = seq_ref[...]
    y = x * s_ref[...]
    e = jnp.exp(jnp.minimum(y, 0.0)) - 1.0
    raw_ref[...] = jnp.where(y > 0.0, y, e) + A4 * seq1_ref[...]


def _stage_a(seq, seq1, svec2d):
    blk = 400
    return pl.pallas_call(
        _ew_body,
        grid=(N // blk,),
        in_specs=[
            pl.BlockSpec((blk, D), lambda i: (i, 0)),
            pl.BlockSpec((blk, D), lambda i: (i, 0)),
            pl.BlockSpec((1, D), lambda i: (0, 0)),
        ],
        out_specs=pl.BlockSpec((blk, D), lambda i: (i, 0)),
        out_shape=jax.ShapeDtypeStruct((N, D), jnp.float32),
    )(seq, seq1, svec2d)


def _sc_body(raw_hbm, lab_hbm,
             sums_out,
             raw0, raw1, lab0, lab1, sums_v,
             sr0, sr1, sl0, sl1):
    wid = lax.axis_index("s") * NC + lax.axis_index("c")
    raws = (raw0, raw1)
    labs = (lab0, lab1)
    srs = (sr0, sr1)
    sls = (sl0, sl1)
    lanes = lax.iota(jnp.int32, 16)
    zero16 = jnp.zeros((16,), jnp.float32)

    for c in range(CP):
        for t in range(D // 16):
            sums_v[c, pl.ds(t * 16, 16)] = zero16

    def issue(i):
        slot = i % 2
        base = (wid + NW * i) * CH
        pltpu.async_copy(raw_hbm.at[pl.ds(base, CH)], raws[slot], srs[slot])
        pltpu.async_copy(lab_hbm.at[pl.ds(base, CH)], labs[slot], sls[slot])

    def wait(i):
        slot = i % 2
        base = (wid + NW * i) * CH
        pltpu.make_async_copy(raw_hbm.at[pl.ds(base, CH)],
                              raws[slot], srs[slot]).wait()
        pltpu.make_async_copy(lab_hbm.at[pl.ds(base, CH)], labs[slot],
                              sls[slot]).wait()

    def process(i):
        slot = i % 2
        raw_v = raws[slot]
        lab_v = labs[slot]

        @plsc.parallel_loop(0, CH, 1, unroll=2)
        def _(r):
            lvec = plsc.load_gather(lab_v.at[...], [jnp.full((16,), r)])
            for j in range(D // 16):
                z = raw_v[r, pl.ds(j * 16, 16)]
                plsc.addupdate_scatter(sums_v.at[...],
                                       [lvec, lanes + j * 16], z)

    issue(0)
    for i in range(MAXK - 1):
        wait(i)
        if i + 1 < MAXK - 1:
            issue(i + 1)
        else:
            @pl.when(wid < _FULL)
            def _():
                issue(MAXK - 1)
        process(i)

    @pl.when(wid < _FULL)
    def _():
        wait(MAXK - 1)
        process(MAXK - 1)

    pltpu.sync_copy(sums_v, sums_out.at[pl.ds(wid * CP, CP)])


_stage_b = functools.partial(
    pl.kernel,
    out_type=[
        jax.ShapeDtypeStruct((NW * CP, D), jnp.float32),
    ],
    mesh=plsc.VectorSubcoreMesh(core_axis_name="c", subcore_axis_name="s"),
    compiler_params=pltpu.CompilerParams(needs_layout_passes=False),
    scratch_types=[
        pltpu.VMEM((CH, D), jnp.float32),
        pltpu.VMEM((CH, D), jnp.float32),
        pltpu.VMEM((CH,), jnp.int32),
        pltpu.VMEM((CH,), jnp.int32),
        pltpu.VMEM((CP, D), jnp.float32),
        pltpu.SemaphoreType.DMA,
        pltpu.SemaphoreType.DMA,
        pltpu.SemaphoreType.DMA,
        pltpu.SemaphoreType.DMA,
    ],
)(_sc_body)


def _head_body(raw_ref, sums_ref, lab_ref, out_ref):
    raw = raw_ref[...]
    s32 = sums_ref[...]
    labs = lab_ref[...]
    sums = s32[0:C, :]
    for w in range(1, NW):
        sums = sums + s32[w * CP:w * CP + C, :]
    ci = lax.broadcasted_iota(jnp.int32, (C, 1), 0)
    counts = jnp.zeros((C, 1), jnp.float32)
    for c in range(C):
        cc = jnp.sum(jnp.where(labs == c, 1.0, 0.0))
        counts = counts + jnp.where(ci == c, cc, 0.0)
    ave = sums / jnp.maximum(counts, 1.0)
    an = jnp.maximum(jnp.sqrt(jnp.sum(ave * ave, axis=1)), 1e-8)
    rn = jnp.maximum(jnp.sqrt(jnp.sum(raw * raw, axis=1, keepdims=True)), 1e-8)
    ret = lax.dot_general(raw, ave, (((1,), (1,)), ((), ())),
                          preferred_element_type=jnp.float32)
    ret = ret / (rn * an[None, :])
    m = jnp.max(ret, axis=1, keepdims=True)
    e = jnp.exp(ret - m)
    out_ref[...] = e / jnp.sum(e, axis=1, keepdims=True)


def _stage_c(raw, sums32, lab2d):
    blk = 1000
    return pl.pallas_call(
        _head_body,
        grid=(N // blk,),
        in_specs=[
            pl.BlockSpec((blk, D), lambda i: (i, 0)),
            pl.BlockSpec((NW * CP, D), lambda i: (0, 0)),
            pl.BlockSpec((N // LBLK, LBLK), lambda i: (0, 0)),
        ],
        out_specs=pl.BlockSpec((blk, C), lambda i: (i, 0)),
        out_shape=jax.ShapeDtypeStruct((N, C), jnp.float32),
    )(raw, sums32, lab2d)


def kernel(seq, seq1, labels, prompt1, prompt2, prompt3,
           wp_weight, dff_weight, dp_weight):
    prompt = jnp.concatenate([prompt1, prompt2, prompt3], axis=0)
    w = 1.0 + jax.nn.elu(jnp.matmul(wp_weight, prompt))
    svec2d = dff_weight[0, 0] * w + dff_weight[0, 1] * dp_weight

    raw = _stage_a(seq, seq1, svec2d)
    (sums32,) = _stage_b(raw, labels)
    return _stage_c(raw, sums32, labels.reshape(N // LBLK, LBLK))


# speedup vs baseline: 4.2316x; 1.1568x over previous
import functools

import jax
import jax.numpy as jnp
from jax import lax
from jax.experimental import pallas as pl
from jax.experimental.pallas import tpu as pltpu
from jax.experimental.pallas import tpu_sc as plsc

N = 10000
D = 256
C = 7
A4 = 0.1

NC = 2
NS = 16
NW = NC * NS
CP = 8
CH = 80
NCHUNK = N // CH
MAXK = -(-NCHUNK // NW)
_FULL = NCHUNK - (NCHUNK // NW) * NW

LBLK = 16


def _ew_body(seq_ref, seq1_ref, p_ref, wp_ref, dff_ref, dp_ref,
             raw_ref, rn2_ref):
    wp0 = wp_ref[0, 0]
    wp1 = wp_ref[0, 1]
    wp2 = wp_ref[0, 2]
    d0 = dff_ref[0, 0]
    d1 = dff_ref[0, 1]
    t = (wp0 * p_ref[0:1, :] + wp1 * p_ref[1:2, :] + wp2 * p_ref[2:3, :])
    te = jnp.exp(jnp.minimum(t, 0.0)) - 1.0
    wvec = 1.0 + jnp.where(t > 0.0, t, te)
    s = d0 * wvec + d1 * dp_ref[...]

    x = seq_ref[...]
    y = x * s
    e = jnp.exp(jnp.minimum(y, 0.0)) - 1.0
    raw = jnp.where(y > 0.0, y, e) + A4 * seq1_ref[...]
    raw_ref[...] = raw
    rn2_ref[...] = jnp.sum(raw * raw, axis=1, keepdims=True)


def _stage_a(seq, seq1, prompt, wp_weight, dff_weight, dp_weight):
    blk = 1000
    return pl.pallas_call(
        _ew_body,
        grid=(N // blk,),
        in_specs=[
            pl.BlockSpec((blk, D), lambda i: (i, 0)),
            pl.BlockSpec((blk, D), lambda i: (i, 0)),
            pl.BlockSpec((3, D), lambda i: (0, 0)),
            pl.BlockSpec((1, 3), lambda i: (0, 0)),
            pl.BlockSpec((1, 2), lambda i: (0, 0)),
            pl.BlockSpec((1, D), lambda i: (0, 0)),
        ],
        out_specs=[
            pl.BlockSpec((blk, D), lambda i: (i, 0)),
            pl.BlockSpec((blk, 1), lambda i: (i, 0)),
        ],
        out_shape=[
            jax.ShapeDtypeStruct((N, D), jnp.float32),
            jax.ShapeDtypeStruct((N, 1), jnp.float32),
        ],
    )(seq, seq1, prompt, wp_weight, dff_weight, dp_weight)


def _sc_body(raw_hbm, lab_hbm,
             sums_out,
             raw0, raw1, lab0, lab1, sums_v,
             sr0, sr1, sl0, sl1):
    wid = lax.axis_index("s") * NC + lax.axis_index("c")
    raws = (raw0, raw1)
    labs = (lab0, lab1)
    srs = (sr0, sr1)
    sls = (sl0, sl1)
    lanes = lax.iota(jnp.int32, 16)
    zero16 = jnp.zeros((16,), jnp.float32)

    for c in range(CP):
        for t in range(D // 16):
            sums_v[c, pl.ds(t * 16, 16)] = zero16

    def issue(i):
        slot = i % 2
        base = (wid + NW * i) * CH
        pltpu.async_copy(raw_hbm.at[pl.ds(base, CH)], raws[slot], srs[slot])
        pltpu.async_copy(lab_hbm.at[pl.ds(base, CH)], labs[slot], sls[slot])

    def wait(i):
        slot = i % 2
        base = (wid + NW * i) * CH
        pltpu.make_async_copy(raw_hbm.at[pl.ds(base, CH)],
                              raws[slot], srs[slot]).wait()
        pltpu.make_async_copy(lab_hbm.at[pl.ds(base, CH)], labs[slot],
                              sls[slot]).wait()

    def process(i):
        slot = i % 2
        raw_v = raws[slot]
        lab_v = labs[slot]

        @plsc.parallel_loop(0, CH, 1, unroll=2)
        def _(r):
            lvec = plsc.load_gather(lab_v.at[...], [jnp.full((16,), r)])
            for j in range(D // 16):
                z = raw_v[r, pl.ds(j * 16, 16)]
                plsc.addupdate_scatter(sums_v.at[...],
                                       [lvec, lanes + j * 16], z)

    issue(0)
    for i in range(MAXK - 1):
        wait(i)
        if i + 1 < MAXK - 1:
            issue(i + 1)
        else:
            @pl.when(wid < _FULL)
            def _():
                issue(MAXK - 1)
        process(i)

    @pl.when(wid < _FULL)
    def _():
        wait(MAXK - 1)
        process(MAXK - 1)

    pltpu.sync_copy(sums_v, sums_out.at[pl.ds(wid * CP, CP)])


_stage_b = functools.partial(
    pl.kernel,
    out_type=[
        jax.ShapeDtypeStruct((NW * CP, D), jnp.float32),
    ],
    mesh=plsc.VectorSubcoreMesh(core_axis_name="c", subcore_axis_name="s"),
    compiler_params=pltpu.CompilerParams(needs_layout_passes=False),
    scratch_types=[
        pltpu.VMEM((CH, D), jnp.float32),
        pltpu.VMEM((CH, D), jnp.float32),
        pltpu.VMEM((CH,), jnp.int32),
        pltpu.VMEM((CH,), jnp.int32),
        pltpu.VMEM((CP, D), jnp.float32),
        pltpu.SemaphoreType.DMA,
        pltpu.SemaphoreType.DMA,
        pltpu.SemaphoreType.DMA,
        pltpu.SemaphoreType.DMA,
    ],
)(_sc_body)


def _head_body(raw_ref, rn2_ref, sums_ref, lab_ref, out_ref, ave_ref):
    @pl.when(pl.program_id(0) == 0)
    def _():
        s32 = sums_ref[...]
        labs = lab_ref[...]
        sums = s32[0:C, :]
        for w in range(1, NW):
            sums = sums + s32[w * CP:w * CP + C, :]
        ci = lax.broadcasted_iota(jnp.int32, (C, 1), 0)
        counts = jnp.zeros((C, 1), jnp.float32)
        for c in range(C):
            cc = jnp.sum(jnp.where(labs == c, 1.0, 0.0))
            counts = counts + jnp.where(ci == c, cc, 0.0)
        ave = sums / jnp.maximum(counts, 1.0)
        an = jnp.maximum(jnp.sqrt(jnp.sum(ave * ave, axis=1, keepdims=True)),
                         1e-8)
        ave_ref[...] = ave / an

    raw = raw_ref[...]
    ave = ave_ref[...]
    rn = jnp.maximum(jnp.sqrt(rn2_ref[...]), 1e-8)
    ret = lax.dot_general(raw, ave, (((1,), (1,)), ((), ())),
                          preferred_element_type=jnp.float32)
    ret = ret / rn
    m = jnp.max(ret, axis=1, keepdims=True)
    e = jnp.exp(ret - m)
    out_ref[...] = e / jnp.sum(e, axis=1, keepdims=True)


def _stage_c(raw, rn2, sums32, lab2d):
    blk = 1000
    return pl.pallas_call(
        _head_body,
        grid=(N // blk,),
        in_specs=[
            pl.BlockSpec((blk, D), lambda i: (i, 0)),
            pl.BlockSpec((blk, 1), lambda i: (i, 0)),
            pl.BlockSpec((NW * CP, D), lambda i: (0, 0)),
            pl.BlockSpec((N // LBLK, LBLK), lambda i: (0, 0)),
        ],
        out_specs=pl.BlockSpec((blk, C), lambda i: (i, 0)),
        out_shape=jax.ShapeDtypeStruct((N, C), jnp.float32),
        scratch_shapes=[
            pltpu.VMEM((C, D), jnp.float32),
        ],
    )(raw, rn2, sums32, lab2d)


def kernel(seq, seq1, labels, prompt1, prompt2, prompt3,
           wp_weight, dff_weight, dp_weight):
    prompt = jnp.concatenate([prompt1, prompt2, prompt3], axis=0)
    raw, rn2 = _stage_a(seq, seq1, prompt, wp_weight, dff_weight, dp_weight)
    (sums32,) = _stage_b(raw, labels)
    return _stage_c(raw, rn2, sums32, labels.reshape(N // LBLK, LBLK))


# speedup vs baseline: 4.6757x; 1.1049x over previous
import functools

import jax
import jax.numpy as jnp
from jax import lax
from jax.experimental import pallas as pl
from jax.experimental.pallas import tpu as pltpu
from jax.experimental.pallas import tpu_sc as plsc

N = 10000
D = 256
C = 7
A4 = 0.1

NC = 2
NS = 16
NW = NC * NS
CP = 8
CH = 80
NCHUNK = N // CH
MAXK = -(-NCHUNK // NW)
_FULL = NCHUNK - (NCHUNK // NW) * NW

LBLK = 16


def _ew_body(seq_ref, seq1_ref, p1_ref, p2_ref, p3_ref, wp_ref, dff_ref,
             dp_ref, raw_ref):
    wp0 = wp_ref[0, 0]
    wp1 = wp_ref[0, 1]
    wp2 = wp_ref[0, 2]
    d0 = dff_ref[0, 0]
    d1 = dff_ref[0, 1]
    t = wp0 * p1_ref[...] + wp1 * p2_ref[...] + wp2 * p3_ref[...]
    te = jnp.exp(jnp.minimum(t, 0.0)) - 1.0
    wvec = 1.0 + jnp.where(t > 0.0, t, te)
    s = d0 * wvec + d1 * dp_ref[...]

    x = seq_ref[...]
    y = x * s
    e = jnp.exp(jnp.minimum(y, 0.0)) - 1.0
    raw_ref[...] = jnp.where(y > 0.0, y, e) + A4 * seq1_ref[...]


def _stage_a(seq, seq1, p1, p2, p3, wp_weight, dff_weight, dp_weight):
    blk = 2000
    row = pl.BlockSpec((1, D), lambda i: (0, 0))
    return pl.pallas_call(
        _ew_body,
        grid=(N // blk,),
        in_specs=[
            pl.BlockSpec((blk, D), lambda i: (i, 0)),
            pl.BlockSpec((blk, D), lambda i: (i, 0)),
            row, row, row,
            pl.BlockSpec((1, 3), lambda i: (0, 0)),
            pl.BlockSpec((1, 2), lambda i: (0, 0)),
            row,
        ],
        out_specs=pl.BlockSpec((blk, D), lambda i: (i, 0)),
        out_shape=jax.ShapeDtypeStruct((N, D), jnp.float32),
    )(seq, seq1, p1, p2, p3, wp_weight, dff_weight, dp_weight)


def _sc_body(raw_hbm, lab_hbm,
             sums_out,
             raw0, raw1, lab0, lab1, sums_v,
             sr0, sr1, sl0, sl1):
    wid = lax.axis_index("s") * NC + lax.axis_index("c")
    raws = (raw0, raw1)
    labs = (lab0, lab1)
    srs = (sr0, sr1)
    sls = (sl0, sl1)
    lanes = lax.iota(jnp.int32, 16)
    zero16 = jnp.zeros((16,), jnp.float32)

    for c in range(CP):
        for t in range(D // 16):
            sums_v[c, pl.ds(t * 16, 16)] = zero16

    def issue(i):
        slot = i % 2
        base = (wid + NW * i) * CH
        pltpu.async_copy(raw_hbm.at[pl.ds(base, CH)], raws[slot], srs[slot])
        pltpu.async_copy(lab_hbm.at[pl.ds(base, CH)], labs[slot], sls[slot])

    def wait(i):
        slot = i % 2
        base = (wid + NW * i) * CH
        pltpu.make_async_copy(raw_hbm.at[pl.ds(base, CH)],
                              raws[slot], srs[slot]).wait()
        pltpu.make_async_copy(lab_hbm.at[pl.ds(base, CH)], labs[slot],
                              sls[slot]).wait()

    def process(i):
        slot = i % 2
        raw_v = raws[slot]
        lab_v = labs[slot]

        @plsc.parallel_loop(0, CH, 1, unroll=2)
        def _(r):
            lvec = plsc.load_gather(lab_v.at[...], [jnp.full((16,), r)])
            for j in range(D // 16):
                z = raw_v[r, pl.ds(j * 16, 16)]
                plsc.addupdate_scatter(sums_v.at[...],
                                       [lvec, lanes + j * 16], z)

    issue(0)
    for i in range(MAXK - 1):
        wait(i)
        if i + 1 < MAXK - 1:
            issue(i + 1)
        else:
            @pl.when(wid < _FULL)
            def _():
                issue(MAXK - 1)
        process(i)

    @pl.when(wid < _FULL)
    def _():
        wait(MAXK - 1)
        process(MAXK - 1)

    pltpu.sync_copy(sums_v, sums_out.at[pl.ds(wid * CP, CP)])


_stage_b = functools.partial(
    pl.kernel,
    out_type=[
        jax.ShapeDtypeStruct((NW * CP, D), jnp.float32),
    ],
    mesh=plsc.VectorSubcoreMesh(core_axis_name="c", subcore_axis_name="s"),
    compiler_params=pltpu.CompilerParams(needs_layout_passes=False),
    scratch_types=[
        pltpu.VMEM((CH, D), jnp.float32),
        pltpu.VMEM((CH, D), jnp.float32),
        pltpu.VMEM((CH,), jnp.int32),
        pltpu.VMEM((CH,), jnp.int32),
        pltpu.VMEM((CP, D), jnp.float32),
        pltpu.SemaphoreType.DMA,
        pltpu.SemaphoreType.DMA,
        pltpu.SemaphoreType.DMA,
        pltpu.SemaphoreType.DMA,
    ],
)(_sc_body)


def _head_body(raw_ref, sums_ref, lab_ref, out_ref, ave_ref):
    @pl.when(pl.program_id(0) == 0)
    def _():
        s32 = sums_ref[...]
        labs = lab_ref[...]
        sums = s32[0:C, :]
        for w in range(1, NW):
            sums = sums + s32[w * CP:w * CP + C, :]
        ci = lax.broadcasted_iota(jnp.int32, (C, 1), 0)
        counts = jnp.zeros((C, 1), jnp.float32)
        for c in range(C):
            cc = jnp.sum(jnp.where(labs == c, 1.0, 0.0))
            counts = counts + jnp.where(ci == c, cc, 0.0)
        ave = sums / jnp.maximum(counts, 1.0)
        an = jnp.maximum(jnp.sqrt(jnp.sum(ave * ave, axis=1, keepdims=True)),
                         1e-8)
        ave_ref[...] = ave / an

    raw = raw_ref[...]
    ave = ave_ref[...]
    rn = jnp.maximum(jnp.sqrt(jnp.sum(raw * raw, axis=1, keepdims=True)),
                     1e-8)
    ret = lax.dot_general(raw, ave, (((1,), (1,)), ((), ())),
                          preferred_element_type=jnp.float32)
    ret = ret / rn
    m = jnp.max(ret, axis=1, keepdims=True)
    e = jnp.exp(ret - m)
    out_ref[...] = e / jnp.sum(e, axis=1, keepdims=True)


def _stage_c(raw, sums32, lab2d):
    blk = 2000
    return pl.pallas_call(
        _head_body,
        grid=(N // blk,),
        in_specs=[
            pl.BlockSpec((blk, D), lambda i: (i, 0)),
            pl.BlockSpec((NW * CP, D), lambda i: (0, 0)),
            pl.BlockSpec((N // LBLK, LBLK), lambda i: (0, 0)),
        ],
        out_specs=pl.BlockSpec((blk, C), lambda i: (i, 0)),
        out_shape=jax.ShapeDtypeStruct((N, C), jnp.float32),
        scratch_shapes=[
            pltpu.VMEM((C, D), jnp.float32),
        ],
    )(raw, sums32, lab2d)


def kernel(seq, seq1, labels, prompt1, prompt2, prompt3,
           wp_weight, dff_weight, dp_weight):
    raw = _stage_a(seq, seq1, prompt1, prompt2, prompt3,
                   wp_weight, dff_weight, dp_weight)
    (sums32,) = _stage_b(raw, labels)
    return _stage_c(raw, sums32, labels.reshape(N // LBLK, LBLK))
